# Initial kernel scaffold; baseline (speedup 1.0000x reference)
import jax
import jax.numpy as jnp
from jax.experimental import pallas as pl


def kernel(tokens, table, W_pool, b_pool, gamma, beta, W_mlp, b_mlp, W_fc1, b_fc1):
    raise NotImplementedError("write your pallas kernel here")


# speedup vs baseline: 45.3583x; 45.3583x over previous
import functools
import math

import jax
import jax.numpy as jnp
from jax import lax
from jax.experimental import pallas as pl
from jax.experimental.pallas import tpu as pltpu
from jax.experimental.pallas import tpu_sc as plsc

_B = 256
_S = 1000
_V = 655
_VPAD = 656
_D = 1024
_NC, _NS, _L = 2, 16, 16
_NW = _NC * _NS
_RW = _B // _NW
_FULL = _S // _L
_TAIL = _S - _FULL * _L
_ZC = _RW * _VPAD // _L


def _hist_body(tok_hbm, out_hbm, tok_v, cnt_v, sem):
    wid = lax.axis_index("s") * _NC + lax.axis_index("c")
    cp = pltpu.make_async_copy(
        tok_hbm.at[pl.ds(wid * _RW * _S, _RW * _S)],
        tok_v.at[pl.ds(0, _RW * _S)], sem)
    cp.start()
    zeros = jnp.zeros((_L,), jnp.float32)
    ones = jnp.ones((_L,), jnp.float32)

    def _zero(k, _):
        cnt_v[pl.ds(k * _L, _L)] = zeros
        return None

    lax.fori_loop(0, _ZC, _zero, None, unroll=8)
    cp.wait()
    tail_mask = lax.iota(jnp.int32, _L) < _TAIL
    for r in range(_RW):
        rbase = jnp.full((_L,), r * _VPAD, jnp.int32)

        def _scat(j, _, r=r, rbase=rbase):
            t = tok_v[pl.ds(r * _S + j * _L, _L)]
            plsc.addupdate_scatter(cnt_v, [rbase + t], ones)
            return None

        lax.fori_loop(0, _FULL, _scat, None, unroll=31)
        t = tok_v[pl.ds(r * _S + _FULL * _L, _L)]
        plsc.addupdate_scatter(cnt_v, [rbase + t], ones, mask=tail_mask)
    pltpu.sync_copy(cnt_v, out_hbm.at[pl.ds(wid * _RW * _VPAD, _RW * _VPAD)])


@functools.partial(
    pl.kernel,
    mesh=plsc.VectorSubcoreMesh(core_axis_name="c", subcore_axis_name="s"),
    out_type=jax.ShapeDtypeStruct((_B * _VPAD,), jnp.float32),
    scratch_types=[
        pltpu.VMEM((_RW * _S + _L,), jnp.int32),
        pltpu.VMEM((_RW * _VPAD,), jnp.float32),
        pltpu.SemaphoreType.DMA,
    ],
    compiler_params=pltpu.CompilerParams(needs_layout_passes=False),
)
def _histogram(tok_hbm, out_hbm, tok_v, cnt_v, sem):
    _hist_body(tok_hbm, out_hbm, tok_v, cnt_v, sem)


_INV_SQRT2 = 1.0 / math.sqrt(2.0)


def _gelu(x):
    return x * 0.5 * (1.0 + lax.erf(x * _INV_SQRT2))


def _dense_body(cnt_ref, tbl_ref, wp_ref, bp_ref, g_ref, be_ref,
                wm_ref, bm_ref, wf_ref, bf_ref, out_ref):
    counts = cnt_ref[...][:, :_V]
    pooled = lax.dot(counts, tbl_ref[...],
                     preferred_element_type=jnp.float32) * (1.0 / _S)
    h = lax.dot(pooled, wp_ref[...],
                preferred_element_type=jnp.float32) + bp_ref[...]
    mu = jnp.mean(h, axis=-1, keepdims=True)
    d = h - mu
    var = jnp.mean(d * d, axis=-1, keepdims=True)
    x = d * lax.rsqrt(var + 1e-5) * g_ref[...] + be_ref[...]
    x = lax.dot(x, wm_ref[...], preferred_element_type=jnp.float32) + bm_ref[...]
    x = _gelu(x)
    x = lax.dot(x, wf_ref[...], preferred_element_type=jnp.float32) + bf_ref[...]
    out_ref[...] = _gelu(x)


def kernel(tokens, table, W_pool, b_pool, gamma, beta, W_mlp, b_mlp, W_fc1, b_fc1):
    tokens = tokens.astype(jnp.int32)
    counts = _histogram(tokens.reshape(-1)).reshape(_B, _VPAD)
    out = pl.pallas_call(
        _dense_body,
        out_shape=jax.ShapeDtypeStruct((_B, 768), jnp.float32),
    )(counts, table, W_pool, b_pool.reshape(1, -1), gamma.reshape(1, -1),
      beta.reshape(1, -1), W_mlp, b_mlp.reshape(1, -1), W_fc1,
      b_fc1.reshape(1, -1))
    return out


# speedup vs baseline: 55.4612x; 1.2227x over previous
import functools
import math

import jax
import jax.numpy as jnp
from jax import lax
from jax.experimental import pallas as pl
from jax.experimental.pallas import tpu as pltpu
from jax.experimental.pallas import tpu_sc as plsc

_B = 256
_S = 1000
_V = 655
_D = 1024
_NCH = 6
_NC, _NS, _L = 2, 16, 16
_NW = _NC * _NS
_RW = _B // _NW
_FULL = _S // _L
_TAIL = _S - _FULL * _L
_CNT = _NCH * _RW * 128
_ZC = _CNT // _L


def _hist_body(tok_hbm, out_hbm, tok_v, cnt_v, sem, osem):
    wid = lax.axis_index("s") * _NC + lax.axis_index("c")
    cp = pltpu.make_async_copy(
        tok_hbm.at[pl.ds(wid * _RW * _S, _RW * _S)],
        tok_v.at[pl.ds(0, _RW * _S)], sem)
    cp.start()
    zeros = jnp.zeros((_L,), jnp.float32)
    ones = jnp.ones((_L,), jnp.float32)

    @functools.partial(plsc.parallel_loop, 0, _ZC, unroll=8)
    def _zero(k):
        cnt_v[pl.ds(k * _L, _L)] = zeros

    cp.wait()
    @functools.partial(plsc.parallel_loop, 0, _RW * _FULL, unroll=8)
    def _scat(i):
        lr = i // _FULL
        t = tok_v[pl.ds(i * _L + _TAIL * lr, _L)]
        idx = t + (t >> 7) * 896 + jnp.full((_L,), lr * 128, jnp.int32)
        plsc.addupdate_scatter(cnt_v, [idx], ones)

    tail_mask = lax.iota(jnp.int32, _L) < _TAIL
    for r in range(_RW):
        t = tok_v[pl.ds(r * _S + _FULL * _L, _L)]
        idx = t + (t >> 7) * 896 + jnp.full((_L,), r * 128, jnp.int32)
        plsc.addupdate_scatter(cnt_v, [idx], ones, mask=tail_mask)
    ocps = [
        pltpu.make_async_copy(
            cnt_v.at[pl.ds(c * _RW * 128, _RW * 128)],
            out_hbm.at[pl.ds((c * _B + _RW * wid) * 128, _RW * 128)], osem)
        for c in range(_NCH)
    ]
    for cp2 in ocps:
        cp2.start()
    for cp2 in ocps:
        cp2.wait()


@functools.partial(
    pl.kernel,
    mesh=plsc.VectorSubcoreMesh(core_axis_name="c", subcore_axis_name="s"),
    out_type=jax.ShapeDtypeStruct((_NCH * _B * 128,), jnp.float32),
    scratch_types=[
        pltpu.VMEM((_RW * _S + _L,), jnp.int32),
        pltpu.VMEM((_CNT,), jnp.float32),
        pltpu.SemaphoreType.DMA,
        pltpu.SemaphoreType.DMA,
    ],
    compiler_params=pltpu.CompilerParams(needs_layout_passes=False),
)
def _histogram(tok_hbm, out_hbm, tok_v, cnt_v, sem, osem):
    _hist_body(tok_hbm, out_hbm, tok_v, cnt_v, sem, osem)


_INV_SQRT2 = 1.0 / math.sqrt(2.0)


def _gelu(x):
    return x * 0.5 * (1.0 + lax.erf(x * _INV_SQRT2))


def _dense_body(cnt_ref, tbl_ref, wp_ref, bp_ref, g_ref, be_ref,
                wm_ref, bm_ref, wf_ref, bf_ref, out_ref):
    pooled = lax.dot(cnt_ref[5 * _B:6 * _B, :_V - 5 * 128],
                     tbl_ref[5 * 128:_V, :], preferred_element_type=jnp.float32)
    for c in range(5):
        pooled += lax.dot(cnt_ref[c * _B:(c + 1) * _B, :],
                          tbl_ref[c * 128:(c + 1) * 128, :],
                          preferred_element_type=jnp.float32)
    pooled = pooled * (1.0 / _S)
    h = lax.dot(pooled, wp_ref[...],
                preferred_element_type=jnp.float32) + bp_ref[...]
    mu = jnp.mean(h, axis=-1, keepdims=True)
    d = h - mu
    var = jnp.mean(d * d, axis=-1, keepdims=True)
    x = d * lax.rsqrt(var + 1e-5) * g_ref[...] + be_ref[...]
    x = lax.dot(x, wm_ref[...], preferred_element_type=jnp.float32) + bm_ref[...]
    x = _gelu(x)
    x = lax.dot(x, wf_ref[...], preferred_element_type=jnp.float32) + bf_ref[...]
    out_ref[...] = _gelu(x)


def kernel(tokens, table, W_pool, b_pool, gamma, beta, W_mlp, b_mlp, W_fc1, b_fc1):
    tokens = tokens.astype(jnp.int32)
    counts = _histogram(tokens.reshape(-1)).reshape(_NCH * _B, 128)
    out = pl.pallas_call(
        _dense_body,
        out_shape=jax.ShapeDtypeStruct((_B, 768), jnp.float32),
    )(counts, table, W_pool, b_pool.reshape(1, -1), gamma.reshape(1, -1),
      beta.reshape(1, -1), W_mlp, b_mlp.reshape(1, -1), W_fc1,
      b_fc1.reshape(1, -1))
    return out
